# baseline (speedup 1.0000x reference)
import functools

import numpy as np
import jax
import jax.numpy as jnp
from jax import lax
from jax.experimental import pallas as pl
from jax.experimental.pallas import tpu as pltpu
from jax.experimental.pallas import tpu_sc as plsc

N_POINTS = 131072
N_LEVELS = 16
F_PER_LEVEL = 2
LOG2_T = 19
T = 2 ** LOG2_T
BASE_RES = 16
PER_LEVEL_SCALE = 1.3819
N_NEURONS = 64
P1 = 2654435761
P2 = 805459861

NC = 2
NS = 16
NW = NC * NS
PTS_PER_W = N_POINTS // NW
C = 512
NCHUNK = PTS_PER_W // C
NGRP = C // 16
ROWS = 8 * C

RES = [int(np.floor(BASE_RES * PER_LEVEL_SCALE ** l)) for l in range(N_LEVELS)]
CORNERS = [(dx, dy, dz) for dx in (0, 1) for dy in (0, 1) for dz in (0, 1)]

DENSE_LVLS = (0, 1)
DENSE_R = [RES[l] + 1 for l in DENSE_LVLS]
DENSE_N = [r ** 3 for r in DENSE_R]
DENSE_NB = [(n + ROWS - 1) // ROWS * ROWS for n in DENSE_N]
DENSE_OFF = [0, DENSE_NB[0]]
DSPAN = sum(DENSE_NB)
FIRST_STREAM_LVL = len(DENSE_LVLS)

_mesh = plsc.VectorSubcoreMesh(
    core_axis_name="c", subcore_axis_name="s", num_cores=NC, num_subcores=NS)

_DNUMS = lax.GatherDimensionNumbers(
    offset_dims=(), collapsed_slice_dims=(0,), start_index_map=(0,))


def _dgather(v, idx):
    return lax.gather(v, idx[:, None], _DNUMS, (1,),
                      mode=lax.GatherScatterMode.PROMISE_IN_BOUNDS)


REPACK_KW = 2048
WORDS_PER_TAB = N_LEVELS * T
REPACK_SPAN = WORDS_PER_TAB // NW
REPACK_ITERS = REPACK_SPAN // REPACK_KW


def _repack_body(p0, p1, p2, p3, o0, o1, o2, o3,
                 inA_v, inB_v, outA_v, outB_v, siA, siB, soA, soB):
    wid = lax.axis_index("s") * NC + lax.axis_index("c")
    ins = (p0, p1, p2, p3)
    outs = (o0, o1, o2, o3)
    bufs = ((inA_v, outA_v, siA, soA), (inB_v, outB_v, siB, soB))

    for e in range(4):
        base_out = wid * REPACK_SPAN

        def fire_in(ci, b, e=e):
            off = base_out + ci * REPACK_KW
            pltpu.async_copy(
                ins[e].at[pl.ds(2 * off, 2 * REPACK_KW)], bufs[b][0], bufs[b][2])

        def compute(in_ref, out_ref):
            def rep_tile(ti, _):
                def rep_vreg(k, _):
                    bo = ti * 256 + k * 16
                    f0 = in_ref[pl.ds(bo, 16)]
                    f1 = in_ref[pl.ds(bo + 128, 16)]
                    packed = plsc.bitcast(
                        plsc.pack(f0, f1, format=plsc.PackFormat.INTERLEAVED),
                        jnp.int32)
                    out_ref[pl.ds(ti * 128 + k * 16, 16)] = packed
                    return 0

                lax.fori_loop(0, 8, rep_vreg, 0)
                return 0

            lax.fori_loop(0, REPACK_KW // 128, rep_tile, 0)

        fire_in(0, 0)
        fire_in(1, 1)

        def rep_pair(cp, _, e=e):
            for b in range(2):
                ci = 2 * cp + b
                in_ref, out_ref, si, so = bufs[b]
                pltpu.make_async_copy(
                    ins[e].at[pl.ds(0, 2 * REPACK_KW)], in_ref, si).wait()

                @pl.when(cp > 0)
                def _():
                    pltpu.make_async_copy(
                        out_ref, outs[e].at[pl.ds(0, REPACK_KW)], so).wait()

                compute(in_ref, out_ref)
                pltpu.async_copy(
                    out_ref, outs[e].at[pl.ds(base_out + ci * REPACK_KW,
                                              REPACK_KW)], so)

                @pl.when(ci + 2 < REPACK_ITERS)
                def _():
                    fire_in(ci + 2, b)
            return 0

        lax.fori_loop(0, REPACK_ITERS // 2, rep_pair, 0)
        for b in range(2):
            pltpu.make_async_copy(
                bufs[b][1], outs[e].at[pl.ds(0, REPACK_KW)], bufs[b][3]).wait()


def _sc_repack(t0, t1, t2, t3):
    f = pl.kernel(
        _repack_body,
        out_type=tuple(
            jax.ShapeDtypeStruct((WORDS_PER_TAB,), jnp.int32)
            for _ in range(4)),
        mesh=_mesh,
        compiler_params=pltpu.CompilerParams(needs_layout_passes=False),
        scratch_types=[
            pltpu.VMEM((2 * REPACK_KW,), jnp.float32),
            pltpu.VMEM((2 * REPACK_KW,), jnp.float32),
            pltpu.VMEM((REPACK_KW,), jnp.int32),
            pltpu.VMEM((REPACK_KW,), jnp.int32),
            pltpu.SemaphoreType.DMA,
            pltpu.SemaphoreType.DMA,
            pltpu.SemaphoreType.DMA,
            pltpu.SemaphoreType.DMA,
        ],
    )
    return f(t0, t1, t2, t3)


def _enc_body(coords, res_hbm, t0, t1, t2, t3, feat_out,
              res_v, xs_v, ys_v, zs_v, idxA_v, idxB_v,
              rowsA_v, rowsB_v, dense_v, slab_v, semA, semB):
    wid = lax.axis_index("s") * NC + lax.axis_index("c")
    tabs = (t0, t1, t2, t3)
    pltpu.sync_copy(res_hbm, res_v)
    iota16 = jax.lax.iota(jnp.int32, 16)

    for e in range(4):
        for li, l in enumerate(DENSE_LVLS):
            rr = DENSE_R[li]
            n = DENSE_N[li]
            base_d = e * DSPAN + DENSE_OFF[li]
            for done in range(0, DENSE_NB[li], ROWS):

                def build_idx(g, _, done=done, rr=rr, n=n, l=l):
                    gid = jnp.minimum(done + g * 16 + iota16, n - 1)
                    gidf = gid.astype(jnp.float32) + 0.5
                    i = (gidf * jnp.float32(1.0 / (rr * rr))).astype(jnp.int32)
                    rem = gid - i * (rr * rr)
                    j = ((rem.astype(jnp.float32) + 0.5)
                         * jnp.float32(1.0 / rr)).astype(jnp.int32)
                    k = rem - j * rr
                    h = (i.astype(jnp.uint32)
                         ^ (j.astype(jnp.uint32) * jnp.uint32(P1))
                         ^ (k.astype(jnp.uint32) * jnp.uint32(P2)))
                    h = h & jnp.uint32(T - 1)
                    idxA_v[pl.ds(g * 16, 16)] = h.astype(jnp.int32) + l * T
                    return 0

                lax.fori_loop(0, ROWS // 16, build_idx, 0)
                pltpu.async_copy(
                    tabs[e].at[idxA_v],
                    dense_v.at[pl.ds(base_d + done, ROWS)], semA).wait()

    def resf_at(l):
        return plsc.load_gather(res_v, [jnp.full((16,), l, dtype=jnp.int32)])

    def phase_a(l, resf, idx_ref):
        def hash_grp(gi, _):
            lbase = T * l
            for gg in range(4):
                g = 4 * gi + gg
                sl = pl.ds(g * 16, 16)
                xv = xs_v[sl] * resf
                yv = ys_v[sl] * resf
                zv = zs_v[sl] * resf
                hx0 = xv.astype(jnp.int32).astype(jnp.uint32)
                hx1 = hx0 + jnp.uint32(1)
                hy0 = yv.astype(jnp.int32).astype(jnp.uint32) * jnp.uint32(P1)
                hy1 = hy0 + jnp.uint32(P1)
                hz0 = zv.astype(jnp.int32).astype(jnp.uint32) * jnp.uint32(P2)
                hz1 = hz0 + jnp.uint32(P2)
                hx = (hx0, hx1)
                hy = (hy0, hy1)
                hz = (hz0, hz1)
                for j, (dx, dy, dz) in enumerate(CORNERS):
                    h = (hx[dx] ^ hy[dy] ^ hz[dz]) & jnp.uint32(T - 1)
                    idx_ref[pl.ds(j * C + g * 16, 16)] = h.astype(jnp.int32) + lbase
            return 0

        lax.fori_loop(0, NGRP // 4, hash_grp, 0)

    def fire(e, idx_ref, rows_ref, sem):
        return pltpu.async_copy(tabs[e].at[idx_ref], rows_ref, sem)

    def wait(e, idx_ref, rows_ref, sem):
        pltpu.make_async_copy(tabs[e].at[idx_ref], rows_ref, sem).wait()

    def phase_b(l, resf, rows_ref):
        def acc_grp(gi, _):
            for gg in range(4):
                g = 4 * gi + gg
                sl = pl.ds(g * 16, 16)
                xv = xs_v[sl] * resf
                yv = ys_v[sl] * resf
                zv = zs_v[sl] * resf
                fx = xv - xv.astype(jnp.int32).astype(jnp.float32)
                fy = yv - yv.astype(jnp.int32).astype(jnp.float32)
                fz = zv - zv.astype(jnp.int32).astype(jnp.float32)
                wx = (1.0 - fx, fx)
                wy = (1.0 - fy, fy)
                wz = (1.0 - fz, fz)
                acc0 = jnp.zeros((16,), dtype=jnp.float32)
                acc1 = jnp.zeros((16,), dtype=jnp.float32)
                c0 = g * 16
                for j, (dx, dy, dz) in enumerate(CORNERS):
                    p = j * C + c0
                    w = wx[dx] * wy[dy] * wz[dz]
                    f0, f1 = plsc.unpack(
                        plsc.bitcast(rows_ref[pl.ds(p, 16)], jnp.bfloat16),
                        format=plsc.PackFormat.INTERLEAVED)
                    acc0 = acc0 + w * f0
                    acc1 = acc1 + w * f1
                slab_v[2 * l, pl.ds(c0, 16)] = acc0
                slab_v[2 * l + 1, pl.ds(c0, 16)] = acc1
            return 0

        lax.fori_loop(0, NGRP // 4, acc_grp, 0)

    def dense_level(e, li):
        l = DENSE_LVLS[li]
        rr = DENSE_R[li]
        resl = jnp.float32(RES[l])
        base_d = e * DSPAN + DENSE_OFF[li]
        s0 = rr * rr

        def dl_grp(gi, _):
            for gg in range(2):
                g = 2 * gi + gg
                sl = pl.ds(g * 16, 16)
                xv = xs_v[sl] * resl
                yv = ys_v[sl] * resl
                zv = zs_v[sl] * resl
                ix = xv.astype(jnp.int32)
                iy = yv.astype(jnp.int32)
                iz = zv.astype(jnp.int32)
                fx = xv - ix.astype(jnp.float32)
                fy = yv - iy.astype(jnp.float32)
                fz = zv - iz.astype(jnp.float32)
                wx = (1.0 - fx, fx)
                wy = (1.0 - fy, fy)
                wz = (1.0 - fz, fz)
                b000 = base_d + (ix * s0 + iy * rr + iz)
                acc0 = jnp.zeros((16,), dtype=jnp.float32)
                acc1 = jnp.zeros((16,), dtype=jnp.float32)
                for (dx, dy, dz) in CORNERS:
                    di = b000 + (dx * s0 + dy * rr + dz)
                    w = wx[dx] * wy[dy] * wz[dz]
                    f0, f1 = plsc.unpack(
                        plsc.bitcast(plsc.load_gather(dense_v, [di]),
                                     jnp.bfloat16),
                        format=plsc.PackFormat.INTERLEAVED)
                    acc0 = acc0 + w * f0
                    acc1 = acc1 + w * f1
                c0 = g * 16
                slab_v[2 * l, pl.ds(c0, 16)] = acc0
                slab_v[2 * l + 1, pl.ds(c0, 16)] = acc1
            return 0

        lax.fori_loop(0, NGRP // 2, dl_grp, 0)

    def chunk_body(ci, _):
        base_pt = wid * PTS_PER_W + ci * C

        for e in range(4):
            pltpu.sync_copy(coords.at[3 * e + 0, pl.ds(base_pt, C)], xs_v)
            pltpu.sync_copy(coords.at[3 * e + 1, pl.ds(base_pt, C)], ys_v)
            pltpu.sync_copy(coords.at[3 * e + 2, pl.ds(base_pt, C)], zs_v)

            L0 = FIRST_STREAM_LVL
            phase_a(L0, resf_at(L0), idxA_v)
            fire(e, idxA_v, rowsA_v, semA)
            phase_a(L0 + 1, resf_at(L0 + 1), idxB_v)
            fire(e, idxB_v, rowsB_v, semB)

            dense_level(e, 0)
            dense_level(e, 1)

            def level_pair(lp, _, e=e):
                for b, (idx_ref, rows_ref, sem) in enumerate(
                        ((idxA_v, rowsA_v, semA), (idxB_v, rowsB_v, semB))):
                    l = FIRST_STREAM_LVL + 2 * lp + b
                    wait(e, idx_ref, rows_ref, sem)
                    phase_b(l, resf_at(l), rows_ref)

                    @pl.when(lp < (N_LEVELS - FIRST_STREAM_LVL) // 2 - 1)
                    def _():
                        l2 = l + 2
                        phase_a(l2, resf_at(l2), idx_ref)
                        fire(e, idx_ref, rows_ref, sem)
                return 0

            lax.fori_loop(0, (N_LEVELS - FIRST_STREAM_LVL) // 2, level_pair, 0)

            pltpu.sync_copy(
                slab_v, feat_out.at[pl.ds(32 * e, 32), pl.ds(base_pt, C)])
        return 0

    lax.fori_loop(0, NCHUNK, chunk_body, 0)


def _sc_encode(coords, res_arr, t0, t1, t2, t3):
    f = pl.kernel(
        _enc_body,
        out_type=jax.ShapeDtypeStruct((128, N_POINTS), jnp.float32),
        mesh=_mesh,
        compiler_params=pltpu.CompilerParams(needs_layout_passes=False),
        scratch_types=[
            pltpu.VMEM((16,), jnp.float32),
            pltpu.VMEM((C,), jnp.float32),
            pltpu.VMEM((C,), jnp.float32),
            pltpu.VMEM((C,), jnp.float32),
            pltpu.VMEM((ROWS,), jnp.int32),
            pltpu.VMEM((ROWS,), jnp.int32),
            pltpu.VMEM((ROWS,), jnp.int32),
            pltpu.VMEM((ROWS,), jnp.int32),
            pltpu.VMEM((4 * DSPAN,), jnp.int32),
            pltpu.VMEM((32, C), jnp.float32),
            pltpu.SemaphoreType.DMA,
            pltpu.SemaphoreType.DMA,
        ],
    )
    return f(coords, res_arr, t0, t1, t2, t3)


def _mlp_body(feat_ref, xyzt_ref, w0a_ref, w0b_ref, w1_ref, w2_ref, out_ref):
    z = feat_ref[...]
    u = xyzt_ref[...]
    h = jnp.dot(w0a_ref[...], z, preferred_element_type=jnp.float32)
    h = h + jnp.dot(w0b_ref[...], u, preferred_element_type=jnp.float32)
    h = jnp.maximum(h, 0.0)
    h = jnp.maximum(jnp.dot(w1_ref[...], h, preferred_element_type=jnp.float32), 0.0)
    out_ref[...] = jnp.dot(w2_ref[...], h, preferred_element_type=jnp.float32)


def _tc_mlp(featT, xyztT, w0aT, w0bT, w1T, w2T):
    bN = 8192
    grid = (N_POINTS // bN,)
    outT = pl.pallas_call(
        _mlp_body,
        grid=grid,
        in_specs=[
            pl.BlockSpec((128, bN), lambda i: (0, i)),
            pl.BlockSpec((4, bN), lambda i: (0, i)),
            pl.BlockSpec((N_NEURONS, 128), lambda i: (0, 0)),
            pl.BlockSpec((N_NEURONS, 4), lambda i: (0, 0)),
            pl.BlockSpec((N_NEURONS, N_NEURONS), lambda i: (0, 0)),
            pl.BlockSpec((1, N_NEURONS), lambda i: (0, 0)),
        ],
        out_specs=pl.BlockSpec((1, bN), lambda i: (0, i)),
        out_shape=jax.ShapeDtypeStruct((1, N_POINTS), jnp.float32),
    )(featT, xyztT, w0aT, w0bT, w1T, w2T)
    return outT.reshape(N_POINTS, 1)


def kernel(x, t, static_table, xyt_table, xzt_table, yzt_table, W0, W1, W2):
    x0 = x[:, 0]
    x1 = x[:, 1]
    x2 = x[:, 2]
    tr = jnp.full((N_POINTS,), t, dtype=jnp.float32)
    coords = jnp.stack(
        [x0, x1, x2,
         x1, x2, tr,
         x0, x2, tr,
         x0, x1, tr])
    res_arr = jnp.asarray(RES, dtype=jnp.float32)
    tabs = [tbl.reshape(N_LEVELS, T // 128, 128, F_PER_LEVEL)
               .transpose(0, 1, 3, 2)
               .reshape(N_LEVELS * T * F_PER_LEVEL)
            for tbl in (static_table, xyt_table, xzt_table, yzt_table)]
    packed = _sc_repack(*tabs)
    featT = _sc_encode(coords, res_arr, *packed)
    xyztT = jnp.stack([x0, x1, x2, tr])
    return _tc_mlp(featT, xyztT, W0[:128].T, W0[128:].T, W1.T, W2.T)


# speedup vs baseline: 1.4472x; 1.4472x over previous
import functools

import numpy as np
import jax
import jax.numpy as jnp
from jax import lax
from jax.experimental import pallas as pl
from jax.experimental.pallas import tpu as pltpu
from jax.experimental.pallas import tpu_sc as plsc

N_POINTS = 131072
N_LEVELS = 16
F_PER_LEVEL = 2
LOG2_T = 19
T = 2 ** LOG2_T
BASE_RES = 16
PER_LEVEL_SCALE = 1.3819
N_NEURONS = 64
P1 = 2654435761
P2 = 805459861

NC = 2
NS = 16
NW = NC * NS
PTS_PER_W = N_POINTS // NW
C = 512
NCHUNK = PTS_PER_W // C
NGRP = C // 16
ROWS = 8 * C

RES = [int(np.floor(BASE_RES * PER_LEVEL_SCALE ** l)) for l in range(N_LEVELS)]
CORNERS = [(dx, dy, dz) for dx in (0, 1) for dy in (0, 1) for dz in (0, 1)]

_mesh = plsc.VectorSubcoreMesh(
    core_axis_name="c", subcore_axis_name="s", num_cores=NC, num_subcores=NS)

_DNUMS = lax.GatherDimensionNumbers(
    offset_dims=(), collapsed_slice_dims=(0,), start_index_map=(0,))


def _dgather(v, idx):
    return lax.gather(v, idx[:, None], _DNUMS, (1,),
                      mode=lax.GatherScatterMode.PROMISE_IN_BOUNDS)


REPACK_KW = 2048
WORDS_PER_TAB = N_LEVELS * T
REPACK_SPAN = WORDS_PER_TAB // NW
REPACK_ITERS = REPACK_SPAN // REPACK_KW


def _repack_body(p0, p1, p2, p3, o0, o1, o2, o3,
                 inA_v, inB_v, outA_v, outB_v, siA, siB, soA, soB):
    wid = lax.axis_index("s") * NC + lax.axis_index("c")
    ins = (p0, p1, p2, p3)
    outs = (o0, o1, o2, o3)
    bufs = ((inA_v, outA_v, siA, soA), (inB_v, outB_v, siB, soB))

    for e in range(4):
        base_out = wid * REPACK_SPAN

        def fire_in(ci, b, e=e):
            off = base_out + ci * REPACK_KW
            pltpu.async_copy(
                ins[e].at[pl.ds(2 * off, 2 * REPACK_KW)], bufs[b][0], bufs[b][2])

        def compute(in_ref, out_ref):
            def rep_tile(ti, _):
                def rep_vreg(k, _):
                    bo = ti * 256 + k * 16
                    f0 = in_ref[pl.ds(bo, 16)]
                    f1 = in_ref[pl.ds(bo + 128, 16)]
                    packed = plsc.bitcast(
                        plsc.pack(f0, f1, format=plsc.PackFormat.INTERLEAVED),
                        jnp.int32)
                    out_ref[pl.ds(ti * 128 + k * 16, 16)] = packed
                    return 0

                lax.fori_loop(0, 8, rep_vreg, 0)
                return 0

            lax.fori_loop(0, REPACK_KW // 128, rep_tile, 0)

        fire_in(0, 0)
        fire_in(1, 1)

        def rep_pair(cp, _, e=e):
            for b in range(2):
                ci = 2 * cp + b
                in_ref, out_ref, si, so = bufs[b]
                pltpu.make_async_copy(
                    ins[e].at[pl.ds(0, 2 * REPACK_KW)], in_ref, si).wait()

                @pl.when(cp > 0)
                def _():
                    pltpu.make_async_copy(
                        out_ref, outs[e].at[pl.ds(0, REPACK_KW)], so).wait()

                compute(in_ref, out_ref)
                pltpu.async_copy(
                    out_ref, outs[e].at[pl.ds(base_out + ci * REPACK_KW,
                                              REPACK_KW)], so)

                @pl.when(ci + 2 < REPACK_ITERS)
                def _():
                    fire_in(ci + 2, b)
            return 0

        lax.fori_loop(0, REPACK_ITERS // 2, rep_pair, 0)
        for b in range(2):
            pltpu.make_async_copy(
                bufs[b][1], outs[e].at[pl.ds(0, REPACK_KW)], bufs[b][3]).wait()


def _sc_repack(t0, t1, t2, t3):
    f = pl.kernel(
        _repack_body,
        out_type=tuple(
            jax.ShapeDtypeStruct((WORDS_PER_TAB,), jnp.int32)
            for _ in range(4)),
        mesh=_mesh,
        compiler_params=pltpu.CompilerParams(needs_layout_passes=False),
        scratch_types=[
            pltpu.VMEM((2 * REPACK_KW,), jnp.float32),
            pltpu.VMEM((2 * REPACK_KW,), jnp.float32),
            pltpu.VMEM((REPACK_KW,), jnp.int32),
            pltpu.VMEM((REPACK_KW,), jnp.int32),
            pltpu.SemaphoreType.DMA,
            pltpu.SemaphoreType.DMA,
            pltpu.SemaphoreType.DMA,
            pltpu.SemaphoreType.DMA,
        ],
    )
    return f(t0, t1, t2, t3)


def _enc_body(coords, res_hbm, t0, t1, t2, t3, feat_out,
              res_v, xs_v, ys_v, zs_v, idxA_v, idxB_v,
              rowsA_v, rowsB_v, slab_v, semA, semB):
    wid = lax.axis_index("s") * NC + lax.axis_index("c")
    tabs = (t0, t1, t2, t3)
    pltpu.sync_copy(res_hbm, res_v)

    def resf_at(l):
        return _dgather(res_v[...], jnp.full((16,), l, dtype=jnp.int32))

    def phase_a(l, resf, idx_ref):
        def hash_grp(gi, _):
            lbase = T * l
            for gg in range(4):
                g = 4 * gi + gg
                sl = pl.ds(g * 16, 16)
                xv = xs_v[sl] * resf
                yv = ys_v[sl] * resf
                zv = zs_v[sl] * resf
                hx0 = xv.astype(jnp.int32).astype(jnp.uint32)
                hx1 = hx0 + jnp.uint32(1)
                hy0 = yv.astype(jnp.int32).astype(jnp.uint32) * jnp.uint32(P1)
                hy1 = hy0 + jnp.uint32(P1)
                hz0 = zv.astype(jnp.int32).astype(jnp.uint32) * jnp.uint32(P2)
                hz1 = hz0 + jnp.uint32(P2)
                hx = (hx0, hx1)
                hy = (hy0, hy1)
                hz = (hz0, hz1)
                for j, (dx, dy, dz) in enumerate(CORNERS):
                    h = (hx[dx] ^ hy[dy] ^ hz[dz]) & jnp.uint32(T - 1)
                    idx_ref[pl.ds(j * C + g * 16, 16)] = h.astype(jnp.int32) + lbase
            return 0

        lax.fori_loop(0, NGRP // 4, hash_grp, 0)

    def fire(e, idx_ref, rows_ref, sem):
        return pltpu.async_copy(tabs[e].at[idx_ref], rows_ref, sem)

    def wait(e, idx_ref, rows_ref, sem):
        pltpu.make_async_copy(tabs[e].at[idx_ref], rows_ref, sem).wait()

    def phase_b(l, resf, rows_ref):
        def acc_grp(gi, _):
            for gg in range(4):
                g = 4 * gi + gg
                sl = pl.ds(g * 16, 16)
                xv = xs_v[sl] * resf
                yv = ys_v[sl] * resf
                zv = zs_v[sl] * resf
                fx = xv - xv.astype(jnp.int32).astype(jnp.float32)
                fy = yv - yv.astype(jnp.int32).astype(jnp.float32)
                fz = zv - zv.astype(jnp.int32).astype(jnp.float32)
                wx = (1.0 - fx, fx)
                wy = (1.0 - fy, fy)
                wz = (1.0 - fz, fz)
                acc0 = jnp.zeros((16,), dtype=jnp.float32)
                acc1 = jnp.zeros((16,), dtype=jnp.float32)
                c0 = g * 16
                for j, (dx, dy, dz) in enumerate(CORNERS):
                    p = j * C + c0
                    w = wx[dx] * wy[dy] * wz[dz]
                    f0, f1 = plsc.unpack(
                        plsc.bitcast(rows_ref[pl.ds(p, 16)], jnp.bfloat16),
                        format=plsc.PackFormat.INTERLEAVED)
                    acc0 = acc0 + w * f0
                    acc1 = acc1 + w * f1
                slab_v[2 * l, pl.ds(c0, 16)] = acc0
                slab_v[2 * l + 1, pl.ds(c0, 16)] = acc1
            return 0

        lax.fori_loop(0, NGRP // 4, acc_grp, 0)

    def chunk_body(ci, _):
        base_pt = wid * PTS_PER_W + ci * C

        for e in range(4):
            pltpu.sync_copy(coords.at[3 * e + 0, pl.ds(base_pt, C)], xs_v)
            pltpu.sync_copy(coords.at[3 * e + 1, pl.ds(base_pt, C)], ys_v)
            pltpu.sync_copy(coords.at[3 * e + 2, pl.ds(base_pt, C)], zs_v)

            phase_a(0, resf_at(0), idxA_v)
            fire(e, idxA_v, rowsA_v, semA)
            phase_a(1, resf_at(1), idxB_v)
            fire(e, idxB_v, rowsB_v, semB)

            def level_pair(lp, _, e=e):
                for b, (idx_ref, rows_ref, sem) in enumerate(
                        ((idxA_v, rowsA_v, semA), (idxB_v, rowsB_v, semB))):
                    l = 2 * lp + b
                    wait(e, idx_ref, rows_ref, sem)
                    phase_b(l, resf_at(l), rows_ref)

                    @pl.when(lp < N_LEVELS // 2 - 1)
                    def _():
                        l2 = l + 2
                        phase_a(l2, resf_at(l2), idx_ref)
                        fire(e, idx_ref, rows_ref, sem)
                return 0

            lax.fori_loop(0, N_LEVELS // 2, level_pair, 0)

            pltpu.sync_copy(
                slab_v, feat_out.at[pl.ds(32 * e, 32), pl.ds(base_pt, C)])
        return 0

    lax.fori_loop(0, NCHUNK, chunk_body, 0)


def _sc_encode(coords, res_arr, t0, t1, t2, t3):
    f = pl.kernel(
        _enc_body,
        out_type=jax.ShapeDtypeStruct((128, N_POINTS), jnp.float32),
        mesh=_mesh,
        compiler_params=pltpu.CompilerParams(needs_layout_passes=False),
        scratch_types=[
            pltpu.VMEM((16,), jnp.float32),
            pltpu.VMEM((C,), jnp.float32),
            pltpu.VMEM((C,), jnp.float32),
            pltpu.VMEM((C,), jnp.float32),
            pltpu.VMEM((ROWS,), jnp.int32),
            pltpu.VMEM((ROWS,), jnp.int32),
            pltpu.VMEM((ROWS,), jnp.int32),
            pltpu.VMEM((ROWS,), jnp.int32),
            pltpu.VMEM((32, C), jnp.float32),
            pltpu.SemaphoreType.DMA,
            pltpu.SemaphoreType.DMA,
        ],
    )
    return f(coords, res_arr, t0, t1, t2, t3)


def _mlp_body(feat_ref, xyzt_ref, w0a_ref, w0b_ref, w1_ref, w2_ref, out_ref):
    z = feat_ref[...]
    u = xyzt_ref[...]
    h = jnp.dot(w0a_ref[...], z, preferred_element_type=jnp.float32)
    h = h + jnp.dot(w0b_ref[...], u, preferred_element_type=jnp.float32)
    h = jnp.maximum(h, 0.0)
    h = jnp.maximum(jnp.dot(w1_ref[...], h, preferred_element_type=jnp.float32), 0.0)
    out_ref[...] = jnp.dot(w2_ref[...], h, preferred_element_type=jnp.float32)


def _tc_mlp(featT, xyztT, w0aT, w0bT, w1T, w2T):
    bN = 8192
    grid = (N_POINTS // bN,)
    outT = pl.pallas_call(
        _mlp_body,
        grid=grid,
        in_specs=[
            pl.BlockSpec((128, bN), lambda i: (0, i)),
            pl.BlockSpec((4, bN), lambda i: (0, i)),
            pl.BlockSpec((N_NEURONS, 128), lambda i: (0, 0)),
            pl.BlockSpec((N_NEURONS, 4), lambda i: (0, 0)),
            pl.BlockSpec((N_NEURONS, N_NEURONS), lambda i: (0, 0)),
            pl.BlockSpec((1, N_NEURONS), lambda i: (0, 0)),
        ],
        out_specs=pl.BlockSpec((1, bN), lambda i: (0, i)),
        out_shape=jax.ShapeDtypeStruct((1, N_POINTS), jnp.float32),
    )(featT, xyztT, w0aT, w0bT, w1T, w2T)
    return outT.reshape(N_POINTS, 1)


def kernel(x, t, static_table, xyt_table, xzt_table, yzt_table, W0, W1, W2):
    x0 = x[:, 0]
    x1 = x[:, 1]
    x2 = x[:, 2]
    tr = jnp.full((N_POINTS,), t, dtype=jnp.float32)
    coords = jnp.stack(
        [x0, x1, x2,
         x1, x2, tr,
         x0, x2, tr,
         x0, x1, tr])
    res_arr = jnp.asarray(RES, dtype=jnp.float32)
    tabs = [tbl.reshape(N_LEVELS, T // 128, 128, F_PER_LEVEL)
               .transpose(0, 1, 3, 2)
               .reshape(N_LEVELS * T * F_PER_LEVEL)
            for tbl in (static_table, xyt_table, xzt_table, yzt_table)]
    packed = _sc_repack(*tabs)
    featT = _sc_encode(coords, res_arr, *packed)
    xyztT = jnp.stack([x0, x1, x2, tr])
    return _tc_mlp(featT, xyztT, W0[:128].T, W0[128:].T, W1.T, W2.T)
